# baseline (speedup 1.0000x reference)
import jax
import jax.numpy as jnp
from jax.experimental import pallas as pl

_BLK = 4096
_BD = 512


def _body(idx_ref, x_ref, pe_ref, o_ref):
    idx = idx_ref[0, 0, :]
    onehot = (idx[:, None] == jax.lax.broadcasted_iota(jnp.int32, (_BLK, 96), 1)
              ).astype(jnp.bfloat16)
    gathered = jnp.dot(onehot, pe_ref[...], preferred_element_type=jnp.float32)
    o_ref[...] = x_ref[...] + gathered


def kernel(x, x_node_inds, pe):
    B, S, D = x.shape
    N = B * S
    x2 = x.reshape(N, D)
    idx2 = jnp.tile(x_node_inds.astype(jnp.int32), B)
    n_blk = N // _BLK
    idx3 = idx2.reshape(n_blk, 1, _BLK)
    pe_pad = jnp.zeros((96, D), jnp.float32).at[:90].set(pe[0]).astype(jnp.bfloat16)

    out2 = pl.pallas_call(
        _body,
        grid=(n_blk, D // _BD),
        in_specs=[
            pl.BlockSpec((1, 1, _BLK), lambda i, j: (i, 0, 0)),
            pl.BlockSpec((_BLK, _BD), lambda i, j: (i, j)),
            pl.BlockSpec((96, _BD), lambda i, j: (0, j)),
        ],
        out_specs=pl.BlockSpec((_BLK, _BD), lambda i, j: (i, j)),
        out_shape=jax.ShapeDtypeStruct((N, D), jnp.float32),
    )(idx3, x2, pe_pad)
    return out2.reshape(B, S, D)


# speedup vs baseline: 1.0158x; 1.0158x over previous
import jax
import jax.numpy as jnp
from jax.experimental import pallas as pl
from jax.experimental.pallas import tpu as pltpu

_BLK = 2048
_BD = 1024


def _body(idx_ref, x_ref, pe_ref, o_ref):
    idx = idx_ref[0, 0, :]
    onehot = (idx[:, None] == jax.lax.broadcasted_iota(jnp.int32, (_BLK, 96), 1)
              ).astype(jnp.bfloat16)
    gathered = jnp.dot(onehot, pe_ref[...], preferred_element_type=jnp.float32)
    o_ref[...] = x_ref[...] + gathered


def kernel(x, x_node_inds, pe):
    B, S, D = x.shape
    N = B * S
    x2 = x.reshape(N, D)
    idx2 = jnp.tile(x_node_inds.astype(jnp.int32), B)
    n_blk = N // _BLK
    idx3 = idx2.reshape(n_blk, 1, _BLK)
    pe_pad = jnp.zeros((96, D), jnp.float32).at[:90].set(pe[0]).astype(jnp.bfloat16)

    out2 = pl.pallas_call(
        _body,
        grid=(n_blk, D // _BD),
        in_specs=[
            pl.BlockSpec((1, 1, _BLK), lambda i, j: (i, 0, 0)),
            pl.BlockSpec((_BLK, _BD), lambda i, j: (i, j)),
            pl.BlockSpec((96, _BD), lambda i, j: (0, j)),
        ],
        out_specs=pl.BlockSpec((_BLK, _BD), lambda i, j: (i, j)),
        out_shape=jax.ShapeDtypeStruct((N, D), jnp.float32),
        compiler_params=pltpu.CompilerParams(
            dimension_semantics=("parallel", "parallel")),
    )(idx3, x2, pe_pad)
    return out2.reshape(B, S, D)
